# baseline (speedup 1.0000x reference)
import functools

import jax
import jax.numpy as jnp
from jax.experimental import pallas as pl

_B = 2048
_K = 512
_V = 64
_G = 64
_NCPG = 8
_T = 128
_GRID = _B // _T


def _vq_kernel(x_ref, e_ref, eT_ref, disc_ref, pen_ref, pen2_ref, ia_ref,
               ig_ref, hist_ref, ent_ref):
    step = pl.program_id(0)
    xt = x_ref[...]
    et = e_ref[...]
    eT = eT_ref[...]

    acc = None
    for c in range(8):
        sqs = [(xt[:, 8 * c + s][:, None] - eT[8 * c + s][None, :]) ** 2
               for s in range(8)]
        t1 = [sqs[s] + sqs[s + 4] for s in range(4)]
        t2 = [t1[s] + t1[s + 2] for s in range(2)]
        t = t2[0] + t2[1]
        acc = t if acc is None else acc + t
    d = jnp.sqrt(acc)

    iota_k = jax.lax.broadcasted_iota(jnp.int32, (_T, _K), 1)

    dmin = jnp.min(d, axis=1, keepdims=True)
    ia = jnp.min(jnp.where(d == dmin, iota_k, _K), axis=1)

    dr = d.reshape(_T, _G, _NCPG)
    g1 = [dr[:, :, s] + dr[:, :, (s + 4) % 8] for s in range(8)]
    g2 = [g1[s] + g1[(s + 2) % 8] for s in range(8)]
    dg = (g2[0] + g2[1]) / 8.0

    iota_g = jax.lax.broadcasted_iota(jnp.int32, (_T, _G), 1)
    gmin = jnp.min(dg, axis=1, keepdims=True)
    ig = jnp.min(jnp.where(dg == gmin, iota_g, _G), axis=1)

    q = 1.0 / d
    mask = (iota_k // _NCPG) == ig[:, None]
    pm = jnp.where(mask, q, 0.0)
    denom = jnp.maximum(jnp.sum(jnp.abs(pm), axis=1, keepdims=True), 1e-12)
    w = pm / denom

    out = jnp.dot(w, et, preferred_element_type=jnp.float32)
    disc_ref[...] = (out - xt) + xt

    diff = xt - out
    s = jnp.sum(diff * diff, axis=1, keepdims=True)
    pen = jnp.sqrt(s) ** 2
    pen_ref[...] = pen
    pen2_ref[...] = pen
    ia_ref[...] = ia[:, None]
    ig_ref[...] = ig[:, None]

    qmax = jnp.max(pm, axis=1, keepdims=True)
    win = jnp.min(jnp.where(pm == qmax, iota_k, _K), axis=1)
    onehot = (jax.lax.broadcasted_iota(jnp.int32, (_T, _K), 1) ==
              win[:, None]).astype(jnp.float32)
    part = jnp.sum(onehot, axis=0, keepdims=True)

    @pl.when(step == 0)
    def _init():
        hist_ref[...] = jnp.zeros_like(hist_ref)
        ent_ref[...] = jnp.zeros_like(ent_ref)

    hist_ref[...] += part

    @pl.when(step == _GRID - 1)
    def _fin():
        hist = hist_ref[...]
        p = hist / _B
        ent = -jnp.sum(jnp.where(hist > 0, p * jnp.log(jnp.where(hist > 0, p, 1.0)), 0.0))
        ent_ref[...] = ent[None, None]


def kernel(x0, embedding0):
    N, S, C, V = x0.shape
    x = x0.reshape(_B, _V)
    e = embedding0.reshape(_K, _V)

    disc, pen, pen2, ia, ig, hist, ent = pl.pallas_call(
        _vq_kernel,
        grid=(_GRID,),
        in_specs=[
            pl.BlockSpec((_T, _V), lambda i: (i, 0)),
            pl.BlockSpec((_K, _V), lambda i: (0, 0)),
            pl.BlockSpec((_V, _K), lambda i: (0, 0)),
        ],
        out_specs=[
            pl.BlockSpec((_T, _V), lambda i: (i, 0)),
            pl.BlockSpec((_T, 1), lambda i: (i, 0)),
            pl.BlockSpec((_T, 1), lambda i: (i, 0)),
            pl.BlockSpec((_T, 1), lambda i: (i, 0)),
            pl.BlockSpec((_T, 1), lambda i: (i, 0)),
            pl.BlockSpec((1, _K), lambda i: (0, 0)),
            pl.BlockSpec((1, 1), lambda i: (0, 0)),
        ],
        out_shape=[
            jax.ShapeDtypeStruct((_B, _V), jnp.float32),
            jax.ShapeDtypeStruct((_B, 1), jnp.float32),
            jax.ShapeDtypeStruct((_B, 1), jnp.float32),
            jax.ShapeDtypeStruct((_B, 1), jnp.int32),
            jax.ShapeDtypeStruct((_B, 1), jnp.int32),
            jax.ShapeDtypeStruct((1, _K), jnp.float32),
            jax.ShapeDtypeStruct((1, 1), jnp.float32),
        ],
    )(x, e, e.T)

    discrete = disc.reshape(N, S, C, V)
    vq_pen = pen.reshape(N, S, C)
    encoder_pen = pen2.reshape(N, S, C)
    entropy = ent[0, 0]
    index_atom_v = ia.reshape(N, S, C)
    index_group_v = ig.reshape(N, S, C)
    return discrete, vq_pen, encoder_pen, entropy, index_atom_v, index_group_v


# speedup vs baseline: 2.8144x; 2.8144x over previous
import functools

import jax
import jax.numpy as jnp
from jax.experimental import pallas as pl

_B = 2048
_K = 512
_V = 64
_G = 64
_NCPG = 8
_T = 256
_GRID = _B // _T


def _vq_kernel(x_ref, e_ref, eT_ref, disc_ref, pen_ref, pen2_ref, ia_ref,
               ig_ref, hist_ref, ent_ref):
    step = pl.program_id(0)
    xt = x_ref[...]
    et = e_ref[...]
    eT = eT_ref[...]

    acc = None
    for c in range(8):
        sqs = [(xt[:, 8 * c + s][:, None] - eT[8 * c + s][None, :]) ** 2
               for s in range(8)]
        t1 = [sqs[s] + sqs[s + 4] for s in range(4)]
        t2 = [t1[s] + t1[s + 2] for s in range(2)]
        t = t2[0] + t2[1]
        acc = t if acc is None else acc + t
    d = jnp.sqrt(acc)

    iota_k = jax.lax.broadcasted_iota(jnp.int32, (_T, _K), 1)

    dmin = jnp.min(d, axis=1, keepdims=True)
    ia = jnp.min(jnp.where(d == dmin, iota_k, _K), axis=1)

    def bfly(v, k):
        a = jnp.concatenate([v[:, k:], v[:, :k]], axis=1)
        b = jnp.concatenate([v[:, -k:], v[:, :-k]], axis=1)
        return jnp.where((iota_k % (2 * k)) < k, a, b)

    t1g = d + bfly(d, 4)
    t2g = t1g + bfly(t1g, 2)
    dgf = (t2g + bfly(t2g, 1)) / 8.0

    grpid = iota_k // _NCPG
    gmin = jnp.min(dgf, axis=1, keepdims=True)
    ig = jnp.min(jnp.where(dgf == gmin, grpid, _G), axis=1)

    q = 1.0 / d
    mask = grpid == ig[:, None]
    pm = jnp.where(mask, q, 0.0)
    denom = jnp.maximum(jnp.sum(jnp.abs(pm), axis=1, keepdims=True), 1e-12)
    w = pm / denom

    out = jnp.dot(w, et, preferred_element_type=jnp.float32)
    disc_ref[...] = (out - xt) + xt

    diff = xt - out
    s = jnp.sum(diff * diff, axis=1, keepdims=True)
    pen = jnp.sqrt(s) ** 2
    pen_ref[...] = pen
    pen2_ref[...] = pen
    ia_ref[...] = ia[:, None]
    ig_ref[...] = ig[:, None]

    qmax = jnp.max(pm, axis=1, keepdims=True)
    win = jnp.min(jnp.where(pm == qmax, iota_k, _K), axis=1)
    onehot = (jax.lax.broadcasted_iota(jnp.int32, (_T, _K), 1) ==
              win[:, None]).astype(jnp.float32)
    part = jnp.sum(onehot, axis=0, keepdims=True)

    @pl.when(step == 0)
    def _init():
        hist_ref[...] = jnp.zeros_like(hist_ref)
        ent_ref[...] = jnp.zeros_like(ent_ref)

    hist_ref[...] += part

    @pl.when(step == _GRID - 1)
    def _fin():
        hist = hist_ref[...]
        p = hist / _B
        ent = -jnp.sum(jnp.where(hist > 0, p * jnp.log(jnp.where(hist > 0, p, 1.0)), 0.0))
        ent_ref[...] = ent[None, None]


def kernel(x0, embedding0):
    N, S, C, V = x0.shape
    x = x0.reshape(_B, _V)
    e = embedding0.reshape(_K, _V)

    disc, pen, pen2, ia, ig, hist, ent = pl.pallas_call(
        _vq_kernel,
        grid=(_GRID,),
        in_specs=[
            pl.BlockSpec((_T, _V), lambda i: (i, 0)),
            pl.BlockSpec((_K, _V), lambda i: (0, 0)),
            pl.BlockSpec((_V, _K), lambda i: (0, 0)),
        ],
        out_specs=[
            pl.BlockSpec((_T, _V), lambda i: (i, 0)),
            pl.BlockSpec((_T, 1), lambda i: (i, 0)),
            pl.BlockSpec((_T, 1), lambda i: (i, 0)),
            pl.BlockSpec((_T, 1), lambda i: (i, 0)),
            pl.BlockSpec((_T, 1), lambda i: (i, 0)),
            pl.BlockSpec((1, _K), lambda i: (0, 0)),
            pl.BlockSpec((1, 1), lambda i: (0, 0)),
        ],
        out_shape=[
            jax.ShapeDtypeStruct((_B, _V), jnp.float32),
            jax.ShapeDtypeStruct((_B, 1), jnp.float32),
            jax.ShapeDtypeStruct((_B, 1), jnp.float32),
            jax.ShapeDtypeStruct((_B, 1), jnp.int32),
            jax.ShapeDtypeStruct((_B, 1), jnp.int32),
            jax.ShapeDtypeStruct((1, _K), jnp.float32),
            jax.ShapeDtypeStruct((1, 1), jnp.float32),
        ],
    )(x, e, e.T)

    discrete = disc.reshape(N, S, C, V)
    vq_pen = pen.reshape(N, S, C)
    encoder_pen = pen2.reshape(N, S, C)
    entropy = ent[0, 0]
    index_atom_v = ia.reshape(N, S, C)
    index_group_v = ig.reshape(N, S, C)
    return discrete, vq_pen, encoder_pen, entropy, index_atom_v, index_group_v


# speedup vs baseline: 2.9273x; 1.0401x over previous
import functools

import jax
import jax.numpy as jnp
from jax.experimental import pallas as pl

_B = 2048
_K = 512
_V = 64
_G = 64
_NCPG = 8
_T = 512
_GRID = _B // _T


def _vq_kernel(x_ref, e_ref, eT_ref, disc_ref, pen_ref, pen2_ref, ia_ref,
               ig_ref, hist_ref, ent_ref):
    step = pl.program_id(0)
    xt = x_ref[...]
    et = e_ref[...]
    eT = eT_ref[...]

    acc = None
    for c in range(8):
        sqs = [(xt[:, 8 * c + s][:, None] - eT[8 * c + s][None, :]) ** 2
               for s in range(8)]
        t1 = [sqs[s] + sqs[s + 4] for s in range(4)]
        t2 = [t1[s] + t1[s + 2] for s in range(2)]
        t = t2[0] + t2[1]
        acc = t if acc is None else acc + t
    d = jnp.sqrt(acc)

    iota_k = jax.lax.broadcasted_iota(jnp.int32, (_T, _K), 1)

    dmin = jnp.min(d, axis=1, keepdims=True)
    ia = jnp.min(jnp.where(d == dmin, iota_k, _K), axis=1)

    def bfly(v, k):
        a = jnp.concatenate([v[:, k:], v[:, :k]], axis=1)
        b = jnp.concatenate([v[:, -k:], v[:, :-k]], axis=1)
        return jnp.where((iota_k % (2 * k)) < k, a, b)

    t1g = d + bfly(d, 4)
    t2g = t1g + bfly(t1g, 2)
    dgf = (t2g + bfly(t2g, 1)) / 8.0

    grpid = iota_k // _NCPG
    gmin = jnp.min(dgf, axis=1, keepdims=True)
    ig = jnp.min(jnp.where(dgf == gmin, grpid, _G), axis=1)

    q = 1.0 / d
    mask = grpid == ig[:, None]
    pm = jnp.where(mask, q, 0.0)
    denom = jnp.maximum(jnp.sum(jnp.abs(pm), axis=1, keepdims=True), 1e-12)
    w = pm / denom

    out = jnp.dot(w, et, preferred_element_type=jnp.float32)
    disc_ref[...] = (out - xt) + xt

    diff = xt - out
    s = jnp.sum(diff * diff, axis=1, keepdims=True)
    pen = jnp.sqrt(s) ** 2
    pen_ref[...] = pen
    pen2_ref[...] = pen
    ia_ref[...] = ia[:, None]
    ig_ref[...] = ig[:, None]

    qmax = jnp.max(pm, axis=1, keepdims=True)
    win = jnp.min(jnp.where(pm == qmax, iota_k, _K), axis=1)
    onehot = (jax.lax.broadcasted_iota(jnp.int32, (_T, _K), 1) ==
              win[:, None]).astype(jnp.float32)
    part = jnp.sum(onehot, axis=0, keepdims=True)

    @pl.when(step == 0)
    def _init():
        hist_ref[...] = jnp.zeros_like(hist_ref)
        ent_ref[...] = jnp.zeros_like(ent_ref)

    hist_ref[...] += part

    @pl.when(step == _GRID - 1)
    def _fin():
        hist = hist_ref[...]
        p = hist / _B
        ent = -jnp.sum(jnp.where(hist > 0, p * jnp.log(jnp.where(hist > 0, p, 1.0)), 0.0))
        ent_ref[...] = ent[None, None]


def kernel(x0, embedding0):
    N, S, C, V = x0.shape
    x = x0.reshape(_B, _V)
    e = embedding0.reshape(_K, _V)

    disc, pen, pen2, ia, ig, hist, ent = pl.pallas_call(
        _vq_kernel,
        grid=(_GRID,),
        in_specs=[
            pl.BlockSpec((_T, _V), lambda i: (i, 0)),
            pl.BlockSpec((_K, _V), lambda i: (0, 0)),
            pl.BlockSpec((_V, _K), lambda i: (0, 0)),
        ],
        out_specs=[
            pl.BlockSpec((_T, _V), lambda i: (i, 0)),
            pl.BlockSpec((_T, 1), lambda i: (i, 0)),
            pl.BlockSpec((_T, 1), lambda i: (i, 0)),
            pl.BlockSpec((_T, 1), lambda i: (i, 0)),
            pl.BlockSpec((_T, 1), lambda i: (i, 0)),
            pl.BlockSpec((1, _K), lambda i: (0, 0)),
            pl.BlockSpec((1, 1), lambda i: (0, 0)),
        ],
        out_shape=[
            jax.ShapeDtypeStruct((_B, _V), jnp.float32),
            jax.ShapeDtypeStruct((_B, 1), jnp.float32),
            jax.ShapeDtypeStruct((_B, 1), jnp.float32),
            jax.ShapeDtypeStruct((_B, 1), jnp.int32),
            jax.ShapeDtypeStruct((_B, 1), jnp.int32),
            jax.ShapeDtypeStruct((1, _K), jnp.float32),
            jax.ShapeDtypeStruct((1, 1), jnp.float32),
        ],
    )(x, e, e.T)

    discrete = disc.reshape(N, S, C, V)
    vq_pen = pen.reshape(N, S, C)
    encoder_pen = pen2.reshape(N, S, C)
    entropy = ent[0, 0]
    index_atom_v = ia.reshape(N, S, C)
    index_group_v = ig.reshape(N, S, C)
    return discrete, vq_pen, encoder_pen, entropy, index_atom_v, index_group_v


# speedup vs baseline: 2.9375x; 1.0035x over previous
import functools

import jax
import jax.numpy as jnp
from jax.experimental import pallas as pl

_B = 2048
_K = 512
_V = 64
_G = 64
_NCPG = 8
_T = 1024
_GRID = _B // _T


def _vq_kernel(x_ref, e_ref, eT_ref, disc_ref, pen_ref, pen2_ref, ia_ref,
               ig_ref, hist_ref, ent_ref):
    step = pl.program_id(0)
    xt = x_ref[...]
    et = e_ref[...]
    eT = eT_ref[...]

    acc = None
    for c in range(8):
        sqs = [(xt[:, 8 * c + s][:, None] - eT[8 * c + s][None, :]) ** 2
               for s in range(8)]
        t1 = [sqs[s] + sqs[s + 4] for s in range(4)]
        t2 = [t1[s] + t1[s + 2] for s in range(2)]
        t = t2[0] + t2[1]
        acc = t if acc is None else acc + t
    d = jnp.sqrt(acc)

    iota_k = jax.lax.broadcasted_iota(jnp.int32, (_T, _K), 1)

    dmin = jnp.min(d, axis=1, keepdims=True)
    ia = jnp.min(jnp.where(d == dmin, iota_k, _K), axis=1)

    def bfly(v, k):
        a = jnp.concatenate([v[:, k:], v[:, :k]], axis=1)
        b = jnp.concatenate([v[:, -k:], v[:, :-k]], axis=1)
        return jnp.where((iota_k % (2 * k)) < k, a, b)

    t1g = d + bfly(d, 4)
    t2g = t1g + bfly(t1g, 2)
    dgf = (t2g + bfly(t2g, 1)) / 8.0

    grpid = iota_k // _NCPG
    gmin = jnp.min(dgf, axis=1, keepdims=True)
    ig = jnp.min(jnp.where(dgf == gmin, grpid, _G), axis=1)

    q = 1.0 / d
    mask = grpid == ig[:, None]
    pm = jnp.where(mask, q, 0.0)
    denom = jnp.maximum(jnp.sum(jnp.abs(pm), axis=1, keepdims=True), 1e-12)
    w = pm / denom

    out = jnp.dot(w, et, preferred_element_type=jnp.float32)
    disc_ref[...] = (out - xt) + xt

    diff = xt - out
    s = jnp.sum(diff * diff, axis=1, keepdims=True)
    pen = jnp.sqrt(s) ** 2
    pen_ref[...] = pen
    pen2_ref[...] = pen
    ia_ref[...] = ia[:, None]
    ig_ref[...] = ig[:, None]

    qmax = jnp.max(pm, axis=1, keepdims=True)
    win = jnp.min(jnp.where(pm == qmax, iota_k, _K), axis=1)
    onehot = (jax.lax.broadcasted_iota(jnp.int32, (_T, _K), 1) ==
              win[:, None]).astype(jnp.float32)
    part = jnp.sum(onehot, axis=0, keepdims=True)

    @pl.when(step == 0)
    def _init():
        hist_ref[...] = jnp.zeros_like(hist_ref)
        ent_ref[...] = jnp.zeros_like(ent_ref)

    hist_ref[...] += part

    @pl.when(step == _GRID - 1)
    def _fin():
        hist = hist_ref[...]
        p = hist / _B
        ent = -jnp.sum(jnp.where(hist > 0, p * jnp.log(jnp.where(hist > 0, p, 1.0)), 0.0))
        ent_ref[...] = ent[None, None]


def kernel(x0, embedding0):
    N, S, C, V = x0.shape
    x = x0.reshape(_B, _V)
    e = embedding0.reshape(_K, _V)

    disc, pen, pen2, ia, ig, hist, ent = pl.pallas_call(
        _vq_kernel,
        grid=(_GRID,),
        in_specs=[
            pl.BlockSpec((_T, _V), lambda i: (i, 0)),
            pl.BlockSpec((_K, _V), lambda i: (0, 0)),
            pl.BlockSpec((_V, _K), lambda i: (0, 0)),
        ],
        out_specs=[
            pl.BlockSpec((_T, _V), lambda i: (i, 0)),
            pl.BlockSpec((_T, 1), lambda i: (i, 0)),
            pl.BlockSpec((_T, 1), lambda i: (i, 0)),
            pl.BlockSpec((_T, 1), lambda i: (i, 0)),
            pl.BlockSpec((_T, 1), lambda i: (i, 0)),
            pl.BlockSpec((1, _K), lambda i: (0, 0)),
            pl.BlockSpec((1, 1), lambda i: (0, 0)),
        ],
        out_shape=[
            jax.ShapeDtypeStruct((_B, _V), jnp.float32),
            jax.ShapeDtypeStruct((_B, 1), jnp.float32),
            jax.ShapeDtypeStruct((_B, 1), jnp.float32),
            jax.ShapeDtypeStruct((_B, 1), jnp.int32),
            jax.ShapeDtypeStruct((_B, 1), jnp.int32),
            jax.ShapeDtypeStruct((1, _K), jnp.float32),
            jax.ShapeDtypeStruct((1, 1), jnp.float32),
        ],
    )(x, e, e.T)

    discrete = disc.reshape(N, S, C, V)
    vq_pen = pen.reshape(N, S, C)
    encoder_pen = pen2.reshape(N, S, C)
    entropy = ent[0, 0]
    index_atom_v = ia.reshape(N, S, C)
    index_group_v = ig.reshape(N, S, C)
    return discrete, vq_pen, encoder_pen, entropy, index_atom_v, index_group_v
